# Initial kernel scaffold; baseline (speedup 1.0000x reference)
import jax
import jax.numpy as jnp
from jax.experimental import pallas as pl


def kernel(in_field, species_idx, scales, shifts):
    raise NotImplementedError("write your pallas kernel here")


# speedup vs baseline: 3.4693x; 3.4693x over previous
import jax
import jax.numpy as jnp
from jax.experimental import pallas as pl
from jax.experimental.pallas import tpu as pltpu

_T = 64
_B = 4000


def _tc_body(idx_ref, tsc_ref, tsh_ref, x_ref, o_ref):
    idxrow = idx_ref[...].reshape(1, -1)
    b = idxrow.shape[1]
    iota0 = jax.lax.broadcasted_iota(jnp.int32, (_T, b), 0)
    onehot_t = (iota0 == idxrow).astype(jnp.float32)
    dn = (((0,), (0,)), ((), ()))
    scale = jax.lax.dot_general(onehot_t, tsc_ref[...], dn,
                                preferred_element_type=jnp.float32)
    shift = jax.lax.dot_general(onehot_t, tsh_ref[...], dn,
                                preferred_element_type=jnp.float32)
    o_ref[...] = scale * x_ref[...] + shift


def kernel(in_field, species_idx, scales, shifts):
    n, d = in_field.shape
    idx3d = species_idx.astype(jnp.int32).reshape(n // _B, 1, _B)
    tsc = jnp.broadcast_to(scales.reshape(_T, 1), (_T, d))
    tsh = jnp.broadcast_to(shifts.reshape(_T, 1), (_T, d))
    return pl.pallas_call(
        _tc_body,
        grid=(n // _B,),
        in_specs=[
            pl.BlockSpec((1, 1, _B), lambda i: (i, 0, 0)),
            pl.BlockSpec((_T, d), lambda i: (0, 0)),
            pl.BlockSpec((_T, d), lambda i: (0, 0)),
            pl.BlockSpec((_B, d), lambda i: (i, 0)),
        ],
        out_specs=pl.BlockSpec((_B, d), lambda i: (i, 0)),
        out_shape=jax.ShapeDtypeStruct((n, d), in_field.dtype),
        compiler_params=pltpu.CompilerParams(
            dimension_semantics=("parallel",),
        ),
    )(idx3d, tsc, tsh, in_field)


# speedup vs baseline: 4.0563x; 1.1692x over previous
import jax
import jax.numpy as jnp
from jax.experimental import pallas as pl
from jax.experimental.pallas import tpu as pltpu

_T = 64
_B = 10000


def _tc_body(idx_ref, tsc_ref, tsh_ref, x_ref, o_ref):
    idxrow = idx_ref[...].reshape(1, -1)
    b = idxrow.shape[1]
    iota0 = jax.lax.broadcasted_iota(jnp.int32, (_T, b), 0)
    onehot_t = (iota0 == idxrow).astype(jnp.float32)
    dn = (((0,), (0,)), ((), ()))
    scale = jax.lax.dot_general(onehot_t, tsc_ref[...], dn,
                                preferred_element_type=jnp.float32)
    shift = jax.lax.dot_general(onehot_t, tsh_ref[...], dn,
                                preferred_element_type=jnp.float32)
    o_ref[...] = scale * x_ref[...] + shift


def kernel(in_field, species_idx, scales, shifts):
    n, d = in_field.shape
    idx3d = species_idx.astype(jnp.int32).reshape(n // _B, 1, _B)
    tsc = jnp.broadcast_to(scales.reshape(_T, 1), (_T, d))
    tsh = jnp.broadcast_to(shifts.reshape(_T, 1), (_T, d))
    return pl.pallas_call(
        _tc_body,
        grid=(n // _B,),
        in_specs=[
            pl.BlockSpec((1, 1, _B), lambda i: (i, 0, 0)),
            pl.BlockSpec((_T, d), lambda i: (0, 0)),
            pl.BlockSpec((_T, d), lambda i: (0, 0)),
            pl.BlockSpec((_B, d), lambda i: (i, 0)),
        ],
        out_specs=pl.BlockSpec((_B, d), lambda i: (i, 0)),
        out_shape=jax.ShapeDtypeStruct((n, d), in_field.dtype),
        compiler_params=pltpu.CompilerParams(
            dimension_semantics=("parallel",),
        ),
    )(idx3d, tsc, tsh, in_field)


# speedup vs baseline: 4.0968x; 1.0100x over previous
import jax
import jax.numpy as jnp
from jax.experimental import pallas as pl
from jax.experimental.pallas import tpu as pltpu

_T = 64
_B = 20000


def _tc_body(idx_ref, tsc_ref, tsh_ref, x_ref, o_ref):
    idxrow = idx_ref[...].reshape(1, -1)
    b = idxrow.shape[1]
    iota0 = jax.lax.broadcasted_iota(jnp.int32, (_T, b), 0)
    onehot_t = (iota0 == idxrow).astype(jnp.float32)
    dn = (((0,), (0,)), ((), ()))
    scale = jax.lax.dot_general(onehot_t, tsc_ref[...], dn,
                                preferred_element_type=jnp.float32)
    shift = jax.lax.dot_general(onehot_t, tsh_ref[...], dn,
                                preferred_element_type=jnp.float32)
    o_ref[...] = scale * x_ref[...] + shift


def kernel(in_field, species_idx, scales, shifts):
    n, d = in_field.shape
    idx3d = species_idx.astype(jnp.int32).reshape(n // _B, 1, _B)
    tsc = jnp.broadcast_to(scales.reshape(_T, 1), (_T, d))
    tsh = jnp.broadcast_to(shifts.reshape(_T, 1), (_T, d))
    return pl.pallas_call(
        _tc_body,
        grid=(n // _B,),
        in_specs=[
            pl.BlockSpec((1, 1, _B), lambda i: (i, 0, 0)),
            pl.BlockSpec((_T, d), lambda i: (0, 0)),
            pl.BlockSpec((_T, d), lambda i: (0, 0)),
            pl.BlockSpec((_B, d), lambda i: (i, 0)),
        ],
        out_specs=pl.BlockSpec((_B, d), lambda i: (i, 0)),
        out_shape=jax.ShapeDtypeStruct((n, d), in_field.dtype),
        compiler_params=pltpu.CompilerParams(
            dimension_semantics=("parallel",),
        ),
    )(idx3d, tsc, tsh, in_field)
